# Initial kernel scaffold; baseline (speedup 1.0000x reference)
import jax
import jax.numpy as jnp
from jax.experimental import pallas as pl


def kernel(input_ids, category_ids, id_table, cat_table, W, b, gamma, beta):
    raise NotImplementedError("write your pallas kernel here")


# speedup vs baseline: 4.2736x; 4.2736x over previous
import functools

import jax
import jax.numpy as jnp
from jax import lax
from jax.experimental import pallas as pl
from jax.experimental.pallas import tpu as pltpu
from jax.experimental.pallas import tpu_sc as plsc

B, L = 4096, 200
VOCAB, CAT = 100000, 1000
D = 128
H = 512
EPS = 1e-12
N = B * L

NC, NS = 2, 16
NW = NC * NS
PER_W = N // NW
CH = 128
STEPS = PER_W // CH

BT = 1024


def _sc_gather_body(ids_hbm, cids_hbm, id_tab, cat_tab, out_id, out_cat,
                    idx1, idx2, rows1, rows2, sem1, sem2):
    wid = lax.axis_index("s") * NC + lax.axis_index("c")
    base_w = wid * PER_W

    def body(i, carry):
        base = base_w + i * CH
        pltpu.sync_copy(ids_hbm.at[pl.ds(base, CH)], idx1)
        pltpu.sync_copy(cids_hbm.at[pl.ds(base, CH)], idx2)
        c1 = pltpu.async_copy(id_tab.at[idx1], rows1, sem1)
        c2 = pltpu.async_copy(cat_tab.at[idx2], rows2, sem2)
        c1.wait()
        c2.wait()
        pltpu.sync_copy(rows1, out_id.at[pl.ds(base, CH)])
        pltpu.sync_copy(rows2, out_cat.at[pl.ds(base, CH)])
        return carry

    lax.fori_loop(0, STEPS, body, 0)


_sc_gather = functools.partial(
    pl.kernel,
    out_type=(
        jax.ShapeDtypeStruct((N, D), jnp.float32),
        jax.ShapeDtypeStruct((N, D), jnp.float32),
    ),
    mesh=plsc.VectorSubcoreMesh(core_axis_name="c", subcore_axis_name="s"),
    scratch_types=[
        pltpu.VMEM((CH,), jnp.int32),
        pltpu.VMEM((CH,), jnp.int32),
        pltpu.VMEM((CH, D), jnp.float32),
        pltpu.VMEM((CH, D), jnp.float32),
        pltpu.SemaphoreType.DMA,
        pltpu.SemaphoreType.DMA,
    ],
)(_sc_gather_body)


def _tc_body(a1_ref, a2_ref, w1_ref, w2_ref, b_ref, g_ref, bt_ref, o_ref):
    y = jnp.dot(a1_ref[...], w1_ref[...], preferred_element_type=jnp.float32)
    y = y + jnp.dot(a2_ref[...], w2_ref[...], preferred_element_type=jnp.float32)
    y = y + b_ref[...]
    mu = jnp.mean(y, axis=-1, keepdims=True)
    d = y - mu
    var = jnp.mean(d * d, axis=-1, keepdims=True)
    o_ref[...] = d * lax.rsqrt(var + EPS) * g_ref[...] + bt_ref[...]


_tc_call = pl.pallas_call(
    _tc_body,
    grid=(N // BT,),
    in_specs=[
        pl.BlockSpec((BT, D), lambda i: (i, 0)),
        pl.BlockSpec((BT, D), lambda i: (i, 0)),
        pl.BlockSpec((D, H), lambda i: (0, 0)),
        pl.BlockSpec((D, H), lambda i: (0, 0)),
        pl.BlockSpec((1, H), lambda i: (0, 0)),
        pl.BlockSpec((1, H), lambda i: (0, 0)),
        pl.BlockSpec((1, H), lambda i: (0, 0)),
    ],
    out_specs=pl.BlockSpec((BT, H), lambda i: (i, 0)),
    out_shape=jax.ShapeDtypeStruct((N, H), jnp.float32),
)


def kernel(input_ids, category_ids, id_table, cat_table, W, b, gamma, beta):
    ids = input_ids.reshape(-1)
    cids = category_ids.reshape(-1)
    id_emb, cat_emb = _sc_gather(ids, cids, id_table, cat_table)
    y = _tc_call(id_emb, cat_emb, W[:D], W[D:],
                 b.reshape(1, H), gamma.reshape(1, H), beta.reshape(1, H))
    return y.reshape(B, L, H)


# speedup vs baseline: 5.1811x; 1.2123x over previous
import functools

import jax
import jax.numpy as jnp
from jax import lax
from jax.experimental import pallas as pl
from jax.experimental.pallas import tpu as pltpu
from jax.experimental.pallas import tpu_sc as plsc

B, L = 4096, 200
VOCAB, CAT = 100000, 1000
D = 128
H = 512
EPS = 1e-12
N = B * L

NCHUNK = 4
NT = N // NCHUNK

NC, NS = 2, 16
NW = NC * NS
PER_W = NT // NW
CH = 128
STEPS = PER_W // CH

BT = 1024


def _sc_gather_body(ids_hbm, cids_hbm, id_tab, cat_tab, out_id, out_cat,
                    idx1, idx2, rows1, rows2, sem1, sem2):
    wid = lax.axis_index("s") * NC + lax.axis_index("c")
    base_w = wid * PER_W

    def body(i, carry):
        base = base_w + i * CH
        pltpu.sync_copy(ids_hbm.at[pl.ds(base, CH)], idx1)
        pltpu.sync_copy(cids_hbm.at[pl.ds(base, CH)], idx2)
        c1 = pltpu.async_copy(id_tab.at[idx1], rows1, sem1)
        c2 = pltpu.async_copy(cat_tab.at[idx2], rows2, sem2)
        c1.wait()
        c2.wait()
        pltpu.sync_copy(rows1, out_id.at[pl.ds(base, CH)])
        pltpu.sync_copy(rows2, out_cat.at[pl.ds(base, CH)])
        return carry

    lax.fori_loop(0, STEPS, body, 0)


_sc_gather = functools.partial(
    pl.kernel,
    out_type=(
        jax.ShapeDtypeStruct((NT, D), jnp.float32),
        jax.ShapeDtypeStruct((NT, D), jnp.float32),
    ),
    mesh=plsc.VectorSubcoreMesh(core_axis_name="c", subcore_axis_name="s"),
    scratch_types=[
        pltpu.VMEM((CH,), jnp.int32),
        pltpu.VMEM((CH,), jnp.int32),
        pltpu.VMEM((CH, D), jnp.float32),
        pltpu.VMEM((CH, D), jnp.float32),
        pltpu.SemaphoreType.DMA,
        pltpu.SemaphoreType.DMA,
    ],
)(_sc_gather_body)


def _tc_body(y_ref, a1_ref, a2_ref, w1_ref, w2_ref, b_ref, g_ref, bt_ref,
             o_ref):
    del y_ref
    y = jnp.dot(a1_ref[...], w1_ref[...], preferred_element_type=jnp.float32)
    y = y + jnp.dot(a2_ref[...], w2_ref[...], preferred_element_type=jnp.float32)
    y = y + b_ref[...]
    mu = jnp.mean(y, axis=-1, keepdims=True)
    d = y - mu
    var = jnp.mean(d * d, axis=-1, keepdims=True)
    o_ref[...] = d * lax.rsqrt(var + EPS) * g_ref[...] + bt_ref[...]


def _tc_body0(a1_ref, a2_ref, w1_ref, w2_ref, b_ref, g_ref, bt_ref, o_ref):
    _tc_body(None, a1_ref, a2_ref, w1_ref, w2_ref, b_ref, g_ref, bt_ref,
             o_ref)


def _make_tc_call(k):
    base = k * (NT // BT)
    return pl.pallas_call(
        _tc_body if k else _tc_body0,
        grid=(NT // BT,),
        in_specs=([pl.BlockSpec(memory_space=pltpu.MemorySpace.HBM)]
                  if k else []) + [
            pl.BlockSpec((BT, D), lambda i: (i, 0)),
            pl.BlockSpec((BT, D), lambda i: (i, 0)),
            pl.BlockSpec((D, H), lambda i: (0, 0)),
            pl.BlockSpec((D, H), lambda i: (0, 0)),
            pl.BlockSpec((1, H), lambda i: (0, 0)),
            pl.BlockSpec((1, H), lambda i: (0, 0)),
            pl.BlockSpec((1, H), lambda i: (0, 0)),
        ],
        out_specs=pl.BlockSpec((BT, H), lambda i, base=base: (base + i, 0)),
        out_shape=jax.ShapeDtypeStruct((N, H), jnp.float32),
        input_output_aliases={0: 0} if k else {},
    )


_tc_calls = [_make_tc_call(k) for k in range(NCHUNK)]


def kernel(input_ids, category_ids, id_table, cat_table, W, b, gamma, beta):
    ids = input_ids.reshape(NCHUNK, NT)
    cids = category_ids.reshape(NCHUNK, NT)
    w1, w2 = W[:D], W[D:]
    b2 = b.reshape(1, H)
    g2 = gamma.reshape(1, H)
    bt2 = beta.reshape(1, H)
    embs = [_sc_gather(ids[k], cids[k], id_table, cat_table)
            for k in range(NCHUNK)]
    ie, ce = embs[0]
    y = _tc_calls[0](ie, ce, w1, w2, b2, g2, bt2)
    for k in range(1, NCHUNK):
        ie, ce = embs[k]
        y = _tc_calls[k](y, ie, ce, w1, w2, b2, g2, bt2)
    return y.reshape(B, L, H)


# speedup vs baseline: 5.7198x; 1.1040x over previous
import functools

import jax
import jax.numpy as jnp
from jax import lax
from jax.experimental import pallas as pl
from jax.experimental.pallas import tpu as pltpu
from jax.experimental.pallas import tpu_sc as plsc

B, L = 4096, 200
VOCAB, CAT = 100000, 1000
D = 128
H = 512
EPS = 1e-12
N = B * L

NCHUNK = 4
NT = N // NCHUNK

NC, NS = 2, 16
NW = NC * NS
PER_W = NT // NW
CH = 128
STEPS = PER_W // CH

BT = 2048


def _sc_gather_body(ids_hbm, cids_hbm, id_tab, cat_tab, out_id, out_cat,
                    idx1, idx2, rows1, rows2, sem1, sem2):
    wid = lax.axis_index("s") * NC + lax.axis_index("c")
    base_w = wid * PER_W

    def body(i, carry):
        base = base_w + i * CH
        pltpu.sync_copy(ids_hbm.at[pl.ds(base, CH)], idx1)
        pltpu.sync_copy(cids_hbm.at[pl.ds(base, CH)], idx2)
        c1 = pltpu.async_copy(id_tab.at[idx1], rows1, sem1)
        c2 = pltpu.async_copy(cat_tab.at[idx2], rows2, sem2)
        c1.wait()
        c2.wait()
        pltpu.sync_copy(rows1, out_id.at[pl.ds(base, CH)])
        pltpu.sync_copy(rows2, out_cat.at[pl.ds(base, CH)])
        return carry

    lax.fori_loop(0, STEPS, body, 0)


_sc_gather = functools.partial(
    pl.kernel,
    out_type=(
        jax.ShapeDtypeStruct((NT, D), jnp.float32),
        jax.ShapeDtypeStruct((NT, D), jnp.float32),
    ),
    mesh=plsc.VectorSubcoreMesh(core_axis_name="c", subcore_axis_name="s"),
    scratch_types=[
        pltpu.VMEM((CH,), jnp.int32),
        pltpu.VMEM((CH,), jnp.int32),
        pltpu.VMEM((CH, D), jnp.float32),
        pltpu.VMEM((CH, D), jnp.float32),
        pltpu.SemaphoreType.DMA,
        pltpu.SemaphoreType.DMA,
    ],
)(_sc_gather_body)


def _tc_body(y_ref, a1_ref, a2_ref, w1_ref, w2_ref, b_ref, g_ref, bt_ref,
             o_ref):
    del y_ref
    y = jnp.dot(a1_ref[...], w1_ref[...], preferred_element_type=jnp.float32)
    y = y + jnp.dot(a2_ref[...], w2_ref[...], preferred_element_type=jnp.float32)
    y = y + b_ref[...]
    mu = jnp.mean(y, axis=-1, keepdims=True)
    d = y - mu
    var = jnp.mean(d * d, axis=-1, keepdims=True)
    o_ref[...] = d * lax.rsqrt(var + EPS) * g_ref[...] + bt_ref[...]


def _tc_body0(a1_ref, a2_ref, w1_ref, w2_ref, b_ref, g_ref, bt_ref, o_ref):
    _tc_body(None, a1_ref, a2_ref, w1_ref, w2_ref, b_ref, g_ref, bt_ref,
             o_ref)


def _make_tc_call(k):
    base = k * (NT // BT)
    return pl.pallas_call(
        _tc_body if k else _tc_body0,
        grid=(NT // BT,),
        in_specs=([pl.BlockSpec(memory_space=pltpu.MemorySpace.HBM)]
                  if k else []) + [
            pl.BlockSpec((BT, D), lambda i: (i, 0)),
            pl.BlockSpec((BT, D), lambda i: (i, 0)),
            pl.BlockSpec((D, H), lambda i: (0, 0)),
            pl.BlockSpec((D, H), lambda i: (0, 0)),
            pl.BlockSpec((1, H), lambda i: (0, 0)),
            pl.BlockSpec((1, H), lambda i: (0, 0)),
            pl.BlockSpec((1, H), lambda i: (0, 0)),
        ],
        out_specs=pl.BlockSpec((BT, H), lambda i, base=base: (base + i, 0)),
        out_shape=jax.ShapeDtypeStruct((N, H), jnp.float32),
        input_output_aliases={0: 0} if k else {},
    )


_tc_calls = [_make_tc_call(k) for k in range(NCHUNK)]


def kernel(input_ids, category_ids, id_table, cat_table, W, b, gamma, beta):
    ids = input_ids.reshape(NCHUNK, NT)
    cids = category_ids.reshape(NCHUNK, NT)
    w1, w2 = W[:D], W[D:]
    b2 = b.reshape(1, H)
    g2 = gamma.reshape(1, H)
    bt2 = beta.reshape(1, H)
    embs = [_sc_gather(ids[k], cids[k], id_table, cat_table)
            for k in range(NCHUNK)]
    ie, ce = embs[0]
    y = _tc_calls[0](ie, ce, w1, w2, b2, g2, bt2)
    for k in range(1, NCHUNK):
        ie, ce = embs[k]
        y = _tc_calls[k](y, ie, ce, w1, w2, b2, g2, bt2)
    return y.reshape(B, L, H)


# speedup vs baseline: 5.9933x; 1.0478x over previous
import functools

import jax
import jax.numpy as jnp
from jax import lax
from jax.experimental import pallas as pl
from jax.experimental.pallas import tpu as pltpu
from jax.experimental.pallas import tpu_sc as plsc

B, L = 4096, 200
VOCAB, CAT = 100000, 1000
D = 128
H = 512
EPS = 1e-12
N = B * L

NCHUNK = 4
NT = N // NCHUNK

NC, NS = 2, 16
NW = NC * NS
PER_W = NT // NW
CH = 128
STEPS = PER_W // CH

BT = 4096


def _sc_gather_body(ids_hbm, cids_hbm, id_tab, cat_tab, out_id, out_cat,
                    idx1, idx2, rows1, rows2, sem1, sem2):
    wid = lax.axis_index("s") * NC + lax.axis_index("c")
    base_w = wid * PER_W

    def body(i, carry):
        base = base_w + i * CH
        pltpu.sync_copy(ids_hbm.at[pl.ds(base, CH)], idx1)
        pltpu.sync_copy(cids_hbm.at[pl.ds(base, CH)], idx2)
        c1 = pltpu.async_copy(id_tab.at[idx1], rows1, sem1)
        c2 = pltpu.async_copy(cat_tab.at[idx2], rows2, sem2)
        c1.wait()
        c2.wait()
        pltpu.sync_copy(rows1, out_id.at[pl.ds(base, CH)])
        pltpu.sync_copy(rows2, out_cat.at[pl.ds(base, CH)])
        return carry

    lax.fori_loop(0, STEPS, body, 0)


_sc_gather = functools.partial(
    pl.kernel,
    out_type=(
        jax.ShapeDtypeStruct((NT, D), jnp.float32),
        jax.ShapeDtypeStruct((NT, D), jnp.float32),
    ),
    mesh=plsc.VectorSubcoreMesh(core_axis_name="c", subcore_axis_name="s"),
    scratch_types=[
        pltpu.VMEM((CH,), jnp.int32),
        pltpu.VMEM((CH,), jnp.int32),
        pltpu.VMEM((CH, D), jnp.float32),
        pltpu.VMEM((CH, D), jnp.float32),
        pltpu.SemaphoreType.DMA,
        pltpu.SemaphoreType.DMA,
    ],
)(_sc_gather_body)


def _tc_body(y_ref, a1_ref, a2_ref, w1_ref, w2_ref, b_ref, g_ref, bt_ref,
             o_ref):
    del y_ref
    y = jnp.dot(a1_ref[...], w1_ref[...], preferred_element_type=jnp.float32)
    y = y + jnp.dot(a2_ref[...], w2_ref[...], preferred_element_type=jnp.float32)
    y = y + b_ref[...]
    mu = jnp.mean(y, axis=-1, keepdims=True)
    d = y - mu
    var = jnp.mean(d * d, axis=-1, keepdims=True)
    o_ref[...] = d * lax.rsqrt(var + EPS) * g_ref[...] + bt_ref[...]


def _tc_body0(a1_ref, a2_ref, w1_ref, w2_ref, b_ref, g_ref, bt_ref, o_ref):
    _tc_body(None, a1_ref, a2_ref, w1_ref, w2_ref, b_ref, g_ref, bt_ref,
             o_ref)


def _make_tc_call(k):
    base = k * (NT // BT)
    return pl.pallas_call(
        _tc_body if k else _tc_body0,
        grid=(NT // BT,),
        in_specs=([pl.BlockSpec(memory_space=pltpu.MemorySpace.HBM)]
                  if k else []) + [
            pl.BlockSpec((BT, D), lambda i: (i, 0)),
            pl.BlockSpec((BT, D), lambda i: (i, 0)),
            pl.BlockSpec((D, H), lambda i: (0, 0)),
            pl.BlockSpec((D, H), lambda i: (0, 0)),
            pl.BlockSpec((1, H), lambda i: (0, 0)),
            pl.BlockSpec((1, H), lambda i: (0, 0)),
            pl.BlockSpec((1, H), lambda i: (0, 0)),
        ],
        out_specs=pl.BlockSpec((BT, H), lambda i, base=base: (base + i, 0)),
        out_shape=jax.ShapeDtypeStruct((N, H), jnp.float32),
        input_output_aliases={0: 0} if k else {},
    )


_tc_calls = [_make_tc_call(k) for k in range(NCHUNK)]


def kernel(input_ids, category_ids, id_table, cat_table, W, b, gamma, beta):
    ids = input_ids.reshape(NCHUNK, NT)
    cids = category_ids.reshape(NCHUNK, NT)
    w1, w2 = W[:D], W[D:]
    b2 = b.reshape(1, H)
    g2 = gamma.reshape(1, H)
    bt2 = beta.reshape(1, H)
    embs = [_sc_gather(ids[k], cids[k], id_table, cat_table)
            for k in range(NCHUNK)]
    ie, ce = embs[0]
    y = _tc_calls[0](ie, ce, w1, w2, b2, g2, bt2)
    for k in range(1, NCHUNK):
        ie, ce = embs[k]
        y = _tc_calls[k](y, ie, ce, w1, w2, b2, g2, bt2)
    return y.reshape(B, L, H)


# speedup vs baseline: 6.0826x; 1.0149x over previous
import functools

import jax
import jax.numpy as jnp
from jax import lax
from jax.experimental import pallas as pl
from jax.experimental.pallas import tpu as pltpu
from jax.experimental.pallas import tpu_sc as plsc

B, L = 4096, 200
VOCAB, CAT = 100000, 1000
D = 128
H = 512
EPS = 1e-12
N = B * L

NCHUNK = 4
NT = N // NCHUNK

NC, NS = 2, 16
NW = NC * NS
PER_W = NT // NW
CH = 128
STEPS = PER_W // CH

BT = 4096


def _sc_gather_body(ids_hbm, cids_hbm, id_tab, cat_tab, out_id, out_cat,
                    idx_a, cidx_a, idx_b, cidx_b,
                    rid_a, rcat_a, rid_b, rcat_b,
                    s_ida, s_cata, s_idb, s_catb):
    wid = lax.axis_index("s") * NC + lax.axis_index("c")
    base_w = wid * PER_W

    def stage(i, idxbuf, cidxbuf):
        b = base_w + i * CH
        pltpu.sync_copy(ids_hbm.at[pl.ds(b, CH)], idxbuf)
        pltpu.sync_copy(cids_hbm.at[pl.ds(b, CH)], cidxbuf)

    def fire(idxbuf, cidxbuf, rid, rcat, sid, scat):
        pltpu.async_copy(id_tab.at[idxbuf], rid, sid)
        pltpu.async_copy(cat_tab.at[cidxbuf], rcat, scat)

    def drain(idxbuf, cidxbuf, rid, rcat, sid, scat):
        pltpu.make_async_copy(id_tab.at[idxbuf], rid, sid).wait()
        pltpu.make_async_copy(cat_tab.at[cidxbuf], rcat, scat).wait()

    def store(i, rid, rcat):
        b = base_w + i * CH
        pltpu.sync_copy(rid, out_id.at[pl.ds(b, CH)])
        pltpu.sync_copy(rcat, out_cat.at[pl.ds(b, CH)])

    stage(0, idx_a, cidx_a)
    fire(idx_a, cidx_a, rid_a, rcat_a, s_ida, s_cata)

    def body(j, carry):
        i0 = 2 * j
        stage(i0 + 1, idx_b, cidx_b)
        fire(idx_b, cidx_b, rid_b, rcat_b, s_idb, s_catb)
        drain(idx_a, cidx_a, rid_a, rcat_a, s_ida, s_cata)
        store(i0, rid_a, rcat_a)

        @pl.when(j < STEPS // 2 - 1)
        def _refill():
            stage(i0 + 2, idx_a, cidx_a)
            fire(idx_a, cidx_a, rid_a, rcat_a, s_ida, s_cata)

        drain(idx_b, cidx_b, rid_b, rcat_b, s_idb, s_catb)
        store(i0 + 1, rid_b, rcat_b)
        return carry

    lax.fori_loop(0, STEPS // 2, body, 0)


_sc_gather = functools.partial(
    pl.kernel,
    out_type=(
        jax.ShapeDtypeStruct((NT, D), jnp.float32),
        jax.ShapeDtypeStruct((NT, D), jnp.float32),
    ),
    mesh=plsc.VectorSubcoreMesh(core_axis_name="c", subcore_axis_name="s"),
    scratch_types=[
        pltpu.VMEM((CH,), jnp.int32),
        pltpu.VMEM((CH,), jnp.int32),
        pltpu.VMEM((CH,), jnp.int32),
        pltpu.VMEM((CH,), jnp.int32),
        pltpu.VMEM((CH, D), jnp.float32),
        pltpu.VMEM((CH, D), jnp.float32),
        pltpu.VMEM((CH, D), jnp.float32),
        pltpu.VMEM((CH, D), jnp.float32),
        pltpu.SemaphoreType.DMA,
        pltpu.SemaphoreType.DMA,
        pltpu.SemaphoreType.DMA,
        pltpu.SemaphoreType.DMA,
    ],
)(_sc_gather_body)


def _tc_body(y_ref, a1_ref, a2_ref, w1_ref, w2_ref, b_ref, g_ref, bt_ref,
             o_ref):
    del y_ref
    y = jnp.dot(a1_ref[...], w1_ref[...], preferred_element_type=jnp.float32)
    y = y + jnp.dot(a2_ref[...], w2_ref[...], preferred_element_type=jnp.float32)
    y = y + b_ref[...]
    mu = jnp.mean(y, axis=-1, keepdims=True)
    d = y - mu
    var = jnp.mean(d * d, axis=-1, keepdims=True)
    o_ref[...] = d * lax.rsqrt(var + EPS) * g_ref[...] + bt_ref[...]


def _tc_body0(a1_ref, a2_ref, w1_ref, w2_ref, b_ref, g_ref, bt_ref, o_ref):
    _tc_body(None, a1_ref, a2_ref, w1_ref, w2_ref, b_ref, g_ref, bt_ref,
             o_ref)


def _make_tc_call(k):
    base = k * (NT // BT)
    return pl.pallas_call(
        _tc_body if k else _tc_body0,
        grid=(NT // BT,),
        in_specs=([pl.BlockSpec(memory_space=pltpu.MemorySpace.HBM)]
                  if k else []) + [
            pl.BlockSpec((BT, D), lambda i: (i, 0)),
            pl.BlockSpec((BT, D), lambda i: (i, 0)),
            pl.BlockSpec((D, H), lambda i: (0, 0)),
            pl.BlockSpec((D, H), lambda i: (0, 0)),
            pl.BlockSpec((1, H), lambda i: (0, 0)),
            pl.BlockSpec((1, H), lambda i: (0, 0)),
            pl.BlockSpec((1, H), lambda i: (0, 0)),
        ],
        out_specs=pl.BlockSpec((BT, H), lambda i, base=base: (base + i, 0)),
        out_shape=jax.ShapeDtypeStruct((N, H), jnp.float32),
        input_output_aliases={0: 0} if k else {},
    )


_tc_calls = [_make_tc_call(k) for k in range(NCHUNK)]


def kernel(input_ids, category_ids, id_table, cat_table, W, b, gamma, beta):
    ids = input_ids.reshape(NCHUNK, NT)
    cids = category_ids.reshape(NCHUNK, NT)
    w1, w2 = W[:D], W[D:]
    b2 = b.reshape(1, H)
    g2 = gamma.reshape(1, H)
    bt2 = beta.reshape(1, H)
    embs = [_sc_gather(ids[k], cids[k], id_table, cat_table)
            for k in range(NCHUNK)]
    ie, ce = embs[0]
    y = _tc_calls[0](ie, ce, w1, w2, b2, g2, bt2)
    for k in range(1, NCHUNK):
        ie, ce = embs[k]
        y = _tc_calls[k](y, ie, ce, w1, w2, b2, g2, bt2)
    return y.reshape(B, L, H)


# speedup vs baseline: 6.1049x; 1.0037x over previous
import functools

import jax
import jax.numpy as jnp
from jax import lax
from jax.experimental import pallas as pl
from jax.experimental.pallas import tpu as pltpu
from jax.experimental.pallas import tpu_sc as plsc

B, L = 4096, 200
VOCAB, CAT = 100000, 1000
D = 128
H = 512
EPS = 1e-12
N = B * L

NCHUNK = 8
NT = N // NCHUNK

NC, NS = 2, 16
NW = NC * NS
PER_W = NT // NW
CH = 128
STEPS = PER_W // CH

BT = 4096


def _sc_gather_body(ids_hbm, cids_hbm, id_tab, cat_tab, out_id, out_cat,
                    idx_a, cidx_a, idx_b, cidx_b,
                    rid_a, rcat_a, rid_b, rcat_b,
                    s_ida, s_cata, s_idb, s_catb):
    wid = lax.axis_index("s") * NC + lax.axis_index("c")
    base_w = wid * PER_W

    def stage(i, idxbuf, cidxbuf):
        b = base_w + i * CH
        pltpu.sync_copy(ids_hbm.at[pl.ds(b, CH)], idxbuf)
        pltpu.sync_copy(cids_hbm.at[pl.ds(b, CH)], cidxbuf)

    def fire(idxbuf, cidxbuf, rid, rcat, sid, scat):
        pltpu.async_copy(id_tab.at[idxbuf], rid, sid)
        pltpu.async_copy(cat_tab.at[cidxbuf], rcat, scat)

    def drain(idxbuf, cidxbuf, rid, rcat, sid, scat):
        pltpu.make_async_copy(id_tab.at[idxbuf], rid, sid).wait()
        pltpu.make_async_copy(cat_tab.at[cidxbuf], rcat, scat).wait()

    def store(i, rid, rcat):
        b = base_w + i * CH
        pltpu.sync_copy(rid, out_id.at[pl.ds(b, CH)])
        pltpu.sync_copy(rcat, out_cat.at[pl.ds(b, CH)])

    stage(0, idx_a, cidx_a)
    fire(idx_a, cidx_a, rid_a, rcat_a, s_ida, s_cata)

    def body(j, carry):
        i0 = 2 * j
        stage(i0 + 1, idx_b, cidx_b)
        fire(idx_b, cidx_b, rid_b, rcat_b, s_idb, s_catb)
        drain(idx_a, cidx_a, rid_a, rcat_a, s_ida, s_cata)
        store(i0, rid_a, rcat_a)

        @pl.when(i0 + 2 < STEPS)
        def _refill():
            stage(i0 + 2, idx_a, cidx_a)
            fire(idx_a, cidx_a, rid_a, rcat_a, s_ida, s_cata)

        drain(idx_b, cidx_b, rid_b, rcat_b, s_idb, s_catb)
        store(i0 + 1, rid_b, rcat_b)
        return carry

    lax.fori_loop(0, STEPS // 2, body, 0)
    if STEPS % 2:
        drain(idx_a, cidx_a, rid_a, rcat_a, s_ida, s_cata)
        store(STEPS - 1, rid_a, rcat_a)


_sc_gather = functools.partial(
    pl.kernel,
    out_type=(
        jax.ShapeDtypeStruct((NT, D), jnp.float32),
        jax.ShapeDtypeStruct((NT, D), jnp.float32),
    ),
    mesh=plsc.VectorSubcoreMesh(core_axis_name="c", subcore_axis_name="s"),
    scratch_types=[
        pltpu.VMEM((CH,), jnp.int32),
        pltpu.VMEM((CH,), jnp.int32),
        pltpu.VMEM((CH,), jnp.int32),
        pltpu.VMEM((CH,), jnp.int32),
        pltpu.VMEM((CH, D), jnp.float32),
        pltpu.VMEM((CH, D), jnp.float32),
        pltpu.VMEM((CH, D), jnp.float32),
        pltpu.VMEM((CH, D), jnp.float32),
        pltpu.SemaphoreType.DMA,
        pltpu.SemaphoreType.DMA,
        pltpu.SemaphoreType.DMA,
        pltpu.SemaphoreType.DMA,
    ],
)(_sc_gather_body)


def _tc_body(y_ref, a1_ref, a2_ref, w1_ref, w2_ref, b_ref, g_ref, bt_ref,
             o_ref):
    del y_ref
    y = jnp.dot(a1_ref[...], w1_ref[...], preferred_element_type=jnp.float32)
    y = y + jnp.dot(a2_ref[...], w2_ref[...], preferred_element_type=jnp.float32)
    y = y + b_ref[...]
    mu = jnp.mean(y, axis=-1, keepdims=True)
    d = y - mu
    var = jnp.mean(d * d, axis=-1, keepdims=True)
    o_ref[...] = d * lax.rsqrt(var + EPS) * g_ref[...] + bt_ref[...]


def _tc_body0(a1_ref, a2_ref, w1_ref, w2_ref, b_ref, g_ref, bt_ref, o_ref):
    _tc_body(None, a1_ref, a2_ref, w1_ref, w2_ref, b_ref, g_ref, bt_ref,
             o_ref)


def _make_tc_call(k):
    base = k * (NT // BT)
    return pl.pallas_call(
        _tc_body if k else _tc_body0,
        grid=(NT // BT,),
        in_specs=([pl.BlockSpec(memory_space=pltpu.MemorySpace.HBM)]
                  if k else []) + [
            pl.BlockSpec((BT, D), lambda i: (i, 0)),
            pl.BlockSpec((BT, D), lambda i: (i, 0)),
            pl.BlockSpec((D, H), lambda i: (0, 0)),
            pl.BlockSpec((D, H), lambda i: (0, 0)),
            pl.BlockSpec((1, H), lambda i: (0, 0)),
            pl.BlockSpec((1, H), lambda i: (0, 0)),
            pl.BlockSpec((1, H), lambda i: (0, 0)),
        ],
        out_specs=pl.BlockSpec((BT, H), lambda i, base=base: (base + i, 0)),
        out_shape=jax.ShapeDtypeStruct((N, H), jnp.float32),
        input_output_aliases={0: 0} if k else {},
    )


_tc_calls = [_make_tc_call(k) for k in range(NCHUNK)]


def kernel(input_ids, category_ids, id_table, cat_table, W, b, gamma, beta):
    ids = input_ids.reshape(NCHUNK, NT)
    cids = category_ids.reshape(NCHUNK, NT)
    w1, w2 = W[:D], W[D:]
    b2 = b.reshape(1, H)
    g2 = gamma.reshape(1, H)
    bt2 = beta.reshape(1, H)
    embs = [_sc_gather(ids[k], cids[k], id_table, cat_table)
            for k in range(NCHUNK)]
    ie, ce = embs[0]
    y = _tc_calls[0](ie, ce, w1, w2, b2, g2, bt2)
    for k in range(1, NCHUNK):
        ie, ce = embs[k]
        y = _tc_calls[k](y, ie, ce, w1, w2, b2, g2, bt2)
    return y.reshape(B, L, H)
